# baseline (speedup 1.0000x reference)
import jax
import jax.numpy as jnp
from jax import lax
from jax.experimental import pallas as pl
from jax.experimental.pallas import tpu as pltpu
from jax.experimental.pallas import tpu_sc as plsc

N = 10000
E = 320000
NF = 9
VOCAB = 119
EMB = 128
HID = 256
NG = 64

NC = 2
NS = 16
NW = NC * NS

_mesh = plsc.VectorSubcoreMesh(core_axis_name="c", subcore_axis_name="s")

QW = 64
KE = 256
EPT = E // NS
NCH = (EPT + KE - 1) // KE
EPAD = NCH * KE - EPT
NBUF = 3
NGRP = NCH // NBUF
NEPI = NCH - NGRP * NBUF
DUMP = N


def _sc_deg_body(dst3d_hbm, degp_hbm, oneb, db2, zb, deg_sh, sd0):
    c = lax.axis_index("c")
    s = lax.axis_index("s")
    for r in range(40):
        zb[pl.ds(r * 16, 16)] = jnp.zeros((16,), jnp.float32)
    r0 = s * 640

    @pl.when(s < 15)
    def _():
        pltpu.sync_copy(zb, deg_sh.at[pl.ds(r0, 640)])

    @pl.when(s == 15)
    def _():
        pltpu.sync_copy(zb.at[pl.ds(0, 400)], deg_sh.at[pl.ds(9600, 400)])

    for r in range(KE // 16):
        oneb[pl.ds(r * 16, 16)] = jnp.ones((16,), jnp.float32)

    pltpu.sync_copy(dst3d_hbm.at[s], db2)
    plsc.subcore_barrier()

    _H = (NCH + 1) // 2
    kbase = c * _H
    ndeg = _H - c * (2 * _H - NCH)

    def deg_fire(j, carry):
        pltpu.async_copy(oneb, deg_sh.at[db2.at[kbase + j]], sd0, add=True)
        return carry

    lax.fori_loop(0, ndeg, deg_fire, 0)

    def deg_drain(j, carry):
        pltpu.make_async_copy(oneb, deg_sh.at[db2.at[kbase]], sd0).wait()
        return carry

    lax.fori_loop(0, ndeg, deg_drain, 0)
    plsc.subcore_barrier()

    @pl.when(s < 15)
    def _():
        pltpu.sync_copy(deg_sh.at[pl.ds(r0, 640)], zb)
        pltpu.sync_copy(zb, degp_hbm.at[pl.ds(c * N + r0, 640)])

    @pl.when(s == 15)
    def _():
        pltpu.sync_copy(deg_sh.at[pl.ds(9600, 400)], zb.at[pl.ds(0, 400)])
        pltpu.sync_copy(zb.at[pl.ds(0, 400)],
                        degp_hbm.at[pl.ds(c * N + 9600, 400)])


_sc_deg = pl.kernel(
    _sc_deg_body,
    out_type=jax.ShapeDtypeStruct((2 * N,), jnp.float32),
    mesh=_mesh,
    scratch_types=[pltpu.VMEM((KE,), jnp.float32),
                   pltpu.VMEM((NCH, KE), jnp.int32),
                   pltpu.VMEM((640,), jnp.float32),
                   pltpu.VMEM_SHARED((N + 16,), jnp.float32),
                   pltpu.SemaphoreType.DMA],
    compiler_params=pltpu.CompilerParams(use_tc_tiling_on_sc=False),
)


def _zero_g0(g0):
    def zrow(r, carry):
        for cc in range(QW // 16):
            g0[r, pl.ds(cc * 16, 16)] = jnp.zeros((16,), jnp.float32)
        return carry

    lax.fori_loop(0, KE, zrow, 0)


def _chunked(base, total, fn):
    off = 0
    while off < total:
        n = min(KE, total - off)
        fn(base + off, n)
        off += n


def _acc_rows(s, fn):
    r0 = s * 640

    @pl.when(s < 15)
    def _():
        _chunked(r0, 640, fn)

    @pl.when(s == 15)
    def _():
        _chunked(9600, 400, fn)


def _agg_pass(q_hbm, out_hbm, sb2, db2, gbufs, gsems, ssems, acc_sh, s):
    g0 = gbufs[0]
    _zero_g0(g0)
    _acc_rows(s, lambda r, n: pltpu.sync_copy(
        g0.at[pl.ds(0, n)], acc_sh.at[pl.ds(r, n)]))
    plsc.subcore_barrier()

    for b in range(NBUF):
        pltpu.async_copy(q_hbm.at[sb2.at[b]], gbufs[b], gsems[b])

    def group(j, carry):
        descs = []
        for b in range(NBUF):
            k = NBUF * j + b
            pltpu.make_async_copy(q_hbm.at[sb2.at[0]], gbufs[b],
                                  gsems[b]).wait()
            descs.append(pltpu.async_copy(
                gbufs[b], acc_sh.at[db2.at[k]], ssems[b], add=True))
        for b in range(NBUF):
            k = NBUF * j + b
            descs[b].wait()

            @pl.when(k + NBUF < NCH)
            def _(b=b, k=k):
                pltpu.async_copy(q_hbm.at[sb2.at[k + NBUF]], gbufs[b],
                                 gsems[b])
        return carry

    lax.fori_loop(0, NGRP, group, 0)
    edescs = []
    for b in range(NEPI):
        pltpu.make_async_copy(q_hbm.at[sb2.at[0]], gbufs[b], gsems[b]).wait()
        edescs.append(pltpu.async_copy(
            gbufs[b], acc_sh.at[db2.at[NGRP * NBUF + b]], ssems[b], add=True))
    for d in edescs:
        d.wait()
    plsc.subcore_barrier()

    def wb(r, n):
        pltpu.sync_copy(acc_sh.at[pl.ds(r, n)], g0.at[pl.ds(0, n)])
        pltpu.sync_copy(g0.at[pl.ds(0, n)], out_hbm.at[pl.ds(r, n)])

    _acc_rows(s, wb)


def _edge_body(src3d_hbm, dst3d_hbm, q0_hbm, q1_hbm, q2_hbm, q3_hbm,
               a0_hbm, a1_hbm, a2_hbm, a3_hbm,
               sb2, db2, g0, g1, g2, acc_sh,
               gs0, gs1, gs2,
               ss0, ss1, ss2):
    c = lax.axis_index("c")
    s = lax.axis_index("s")
    gbufs = (g0, g1, g2)
    gsems = (gs0, gs1, gs2)
    ssems = (ss0, ss1, ss2)
    pltpu.sync_copy(src3d_hbm.at[s], sb2)
    pltpu.sync_copy(dst3d_hbm.at[s], db2)

    @pl.when(c == 0)
    def _():
        _agg_pass(q0_hbm, a0_hbm, sb2, db2, gbufs, gsems, ssems, acc_sh, s)
        plsc.subcore_barrier()
        _agg_pass(q1_hbm, a1_hbm, sb2, db2, gbufs, gsems, ssems, acc_sh, s)

    @pl.when(c == 1)
    def _():
        _agg_pass(q2_hbm, a2_hbm, sb2, db2, gbufs, gsems, ssems, acc_sh, s)
        plsc.subcore_barrier()
        _agg_pass(q3_hbm, a3_hbm, sb2, db2, gbufs, gsems, ssems, acc_sh, s)


_edge_scratch = [pltpu.VMEM((NCH, KE), jnp.int32),
                 pltpu.VMEM((NCH, KE), jnp.int32)] + \
                [pltpu.VMEM((KE, QW), jnp.float32)] * NBUF + \
                [pltpu.VMEM_SHARED((N + 16, QW), jnp.float32)] + \
                [pltpu.SemaphoreType.DMA] * (2 * NBUF)

_edge_agg = pl.kernel(
    _edge_body,
    out_type=[jax.ShapeDtypeStruct((N, QW), jnp.float32)] * 4,
    mesh=_mesh,
    scratch_types=_edge_scratch,
    compiler_params=pltpu.CompilerParams(use_tc_tiling_on_sc=False),
)


BLK = 1000
NB = N // BLK


def _prescale_body(x_ref, emb_ref, dpT_ref, s_ref, qa_ref, qb_ref):
    vids = lax.broadcasted_iota(jnp.int32, (BLK, 128), 1)
    h = jnp.zeros((BLK, EMB), jnp.float32)
    for f in range(NF):
        oh = (x_ref[:, f:f + 1] == vids).astype(jnp.float32)
        h = h + jnp.dot(oh, emb_ref[f])
    d = dpT_ref[:, 0:1] + dpT_ref[:, 1:2]
    s = lax.rsqrt(1.0 + d)
    s_ref[...] = s
    q = h * s
    qa_ref[...] = q[:, :QW]
    qb_ref[...] = q[:, QW:]


def _tc_prescale(x, emb_pad, dpT):
    return pl.pallas_call(
        _prescale_body,
        grid=(NB,),
        in_specs=[pl.BlockSpec((BLK, NF), lambda i: (i, 0)),
                  pl.BlockSpec((NF, 128, EMB), lambda i: (0, 0, 0)),
                  pl.BlockSpec((BLK, 2), lambda i: (i, 0))],
        out_specs=[pl.BlockSpec((BLK, 1), lambda i: (i, 0)),
                   pl.BlockSpec((BLK, QW), lambda i: (i, 0)),
                   pl.BlockSpec((BLK, QW), lambda i: (i, 0))],
        out_shape=[jax.ShapeDtypeStruct((N, 1), jnp.float32),
                   jax.ShapeDtypeStruct((N, QW), jnp.float32),
                   jax.ShapeDtypeStruct((N, QW), jnp.float32)],
    )(x, emb_pad, dpT)


def _layer0_body(a0, a1, q0, q1, s_ref, w_ref, b_ref, o0, o1, o2, o3):
    s = s_ref[...]
    z = jnp.concatenate([a0[...] + q0[...], a1[...] + q1[...]], axis=1) * s
    h = jnp.maximum(jnp.dot(z, w_ref[...]) + b_ref[...], 0.0)
    qn = h * s
    o0[...] = qn[:, 0 * QW:1 * QW]
    o1[...] = qn[:, 1 * QW:2 * QW]
    o2[...] = qn[:, 2 * QW:3 * QW]
    o3[...] = qn[:, 3 * QW:4 * QW]


def _layer_body(a0, a1, a2, a3, q0, q1, q2, q3, s_ref, w_ref, b_ref,
                o0, o1, o2, o3):
    s = s_ref[...]
    z = jnp.concatenate([a0[...] + q0[...], a1[...] + q1[...],
                         a2[...] + q2[...], a3[...] + q3[...]], axis=1) * s
    h = jnp.maximum(jnp.dot(z, w_ref[...]) + b_ref[...], 0.0)
    qn = h * s
    o0[...] = qn[:, 0 * QW:1 * QW]
    o1[...] = qn[:, 1 * QW:2 * QW]
    o2[...] = qn[:, 2 * QW:3 * QW]
    o3[...] = qn[:, 3 * QW:4 * QW]


_qspec = pl.BlockSpec((BLK, QW), lambda i: (i, 0))


def _tc_layer0(aq, qq, s, wm, bias):
    return pl.pallas_call(
        _layer0_body,
        grid=(NB,),
        in_specs=[_qspec] * 4 + [
            pl.BlockSpec((BLK, 1), lambda i: (i, 0)),
            pl.BlockSpec((EMB, HID), lambda i: (0, 0)),
            pl.BlockSpec((1, HID), lambda i: (0, 0))],
        out_specs=[_qspec] * 4,
        out_shape=[jax.ShapeDtypeStruct((N, QW), jnp.float32)] * 4,
    )(*aq, *qq, s, wm, bias.reshape(1, HID))


def _tc_layer(aq, qq, s, wm, bias):
    return pl.pallas_call(
        _layer_body,
        grid=(NB,),
        in_specs=[_qspec] * 8 + [
            pl.BlockSpec((BLK, 1), lambda i: (i, 0)),
            pl.BlockSpec((HID, HID), lambda i: (0, 0)),
            pl.BlockSpec((1, HID), lambda i: (0, 0))],
        out_specs=[_qspec] * 4,
        out_shape=[jax.ShapeDtypeStruct((N, QW), jnp.float32)] * 4,
    )(*aq, *qq, s, wm, bias.reshape(1, HID))


def _readout_body(q0, q1, q2, q3, s_ref, bidx_ref, linw_ref, linb_ref,
                  out_ref, sums_scr, cnts_scr):
    i = pl.program_id(0)
    s = s_ref[...]
    h3 = jnp.concatenate([q0[...], q1[...], q2[...], q3[...]], axis=1) / s
    gids = lax.broadcasted_iota(jnp.int32, (NG, BLK), 0)
    ohT = (bidx_ref[0] == gids).astype(jnp.float32)
    bs = jnp.dot(ohT, h3)
    bc = jnp.sum(ohT, axis=1, keepdims=True)

    @pl.when(i == 0)
    def _():
        sums_scr[...] = bs
        cnts_scr[...] = bc

    @pl.when(i > 0)
    def _():
        sums_scr[...] += bs
        cnts_scr[...] += bc

    @pl.when(i == NB - 1)
    def _():
        mean = sums_scr[...] / jnp.maximum(cnts_scr[...], 1.0)
        out_ref[...] = jax.nn.sigmoid(jnp.dot(mean, linw_ref[...])
                                      + linb_ref[...])


def _tc_readout(qq, s, bidx_3d, lin_W, lin_b):
    return pl.pallas_call(
        _readout_body,
        grid=(NB,),
        in_specs=[_qspec] * 4 + [
            pl.BlockSpec((BLK, 1), lambda i: (i, 0)),
            pl.BlockSpec((1, 1, BLK), lambda i: (i, 0, 0)),
            pl.BlockSpec((HID, 1), lambda i: (0, 0)),
            pl.BlockSpec((1, 1), lambda i: (0, 0))],
        out_specs=pl.BlockSpec((NG, 1), lambda i: (0, 0)),
        out_shape=jax.ShapeDtypeStruct((NG, 1), jnp.float32),
        scratch_shapes=[pltpu.VMEM((NG, HID), jnp.float32),
                        pltpu.VMEM((NG, 1), jnp.float32)],
    )(*qq, s, bidx_3d, lin_W, lin_b.reshape(1, 1))


def kernel(x, edge_index, batch_idx, atom_emb, W0, b0, W1, b1, W2, b2,
           lin_W, lin_b):
    x = x.astype(jnp.int32)
    src = edge_index[0].astype(jnp.int32)
    dst = edge_index[1].astype(jnp.int32)
    emb_pad = jnp.pad(atom_emb, ((0, 0), (0, 128 - VOCAB), (0, 0)))

    srcr = src.reshape(NS, EPT)
    dstr = dst.reshape(NS, EPT)
    src3d = jnp.concatenate(
        [srcr, jnp.zeros((NS, EPAD), jnp.int32)], axis=1).reshape(NS, NCH, KE)
    dst3d = jnp.concatenate(
        [dstr, jnp.full((NS, EPAD), DUMP, jnp.int32)],
        axis=1).reshape(NS, NCH, KE)

    degp = _sc_deg(dst3d)
    s, q0a, q0b = _tc_prescale(x, emb_pad, jnp.transpose(degp.reshape(2, N)))

    zq = jnp.zeros((N, QW), jnp.float32)
    aq = _edge_agg(src3d, dst3d, q0a, q0b, zq, zq)
    carry = _tc_layer0(aq[:2], (q0a, q0b), s, W0, b0)
    for wm, bias in ((W1, b1), (W2, b2)):
        aq = _edge_agg(src3d, dst3d, *carry)
        carry = _tc_layer(aq, carry, s, wm, bias)
    q3 = carry

    out = _tc_readout(q3, s,
                      batch_idx.astype(jnp.int32).reshape(NB, 1, BLK),
                      lin_W, lin_b)
    return out


# speedup vs baseline: 1.2887x; 1.2887x over previous
import jax
import jax.numpy as jnp
from jax import lax
from jax.experimental import pallas as pl
from jax.experimental.pallas import tpu as pltpu
from jax.experimental.pallas import tpu_sc as plsc

N = 10000
E = 320000
NF = 9
VOCAB = 119
EMB = 128
HID = 256
NG = 64

NC = 2
NS = 16
NW = NC * NS

_mesh = plsc.VectorSubcoreMesh(core_axis_name="c", subcore_axis_name="s")

QW = 64
KE = 128
EPT = E // NS
NCH = (EPT + KE - 1) // KE
EPAD = NCH * KE - EPT
NBUF = 6
NGRP = NCH // NBUF
NEPI = NCH - NGRP * NBUF
DUMP = N


def _sc_deg_body(dst3d_hbm, degp_hbm, oneb, db2, zb, deg_sh, sd0):
    c = lax.axis_index("c")
    s = lax.axis_index("s")
    for r in range(40):
        zb[pl.ds(r * 16, 16)] = jnp.zeros((16,), jnp.float32)
    r0 = s * 640

    @pl.when(s < 15)
    def _():
        pltpu.sync_copy(zb, deg_sh.at[pl.ds(r0, 640)])

    @pl.when(s == 15)
    def _():
        pltpu.sync_copy(zb.at[pl.ds(0, 400)], deg_sh.at[pl.ds(9600, 400)])

    for r in range(KE // 16):
        oneb[pl.ds(r * 16, 16)] = jnp.ones((16,), jnp.float32)

    pltpu.sync_copy(dst3d_hbm.at[s], db2)
    plsc.subcore_barrier()

    _H = (NCH + 1) // 2
    kbase = c * _H
    ndeg = _H - c * (2 * _H - NCH)

    def deg_fire(j, carry):
        pltpu.async_copy(oneb, deg_sh.at[db2.at[kbase + j]], sd0, add=True)
        return carry

    lax.fori_loop(0, ndeg, deg_fire, 0)

    def deg_drain(j, carry):
        pltpu.make_async_copy(oneb, deg_sh.at[db2.at[kbase]], sd0).wait()
        return carry

    lax.fori_loop(0, ndeg, deg_drain, 0)
    plsc.subcore_barrier()

    @pl.when(s < 15)
    def _():
        pltpu.sync_copy(deg_sh.at[pl.ds(r0, 640)], zb)
        pltpu.sync_copy(zb, degp_hbm.at[pl.ds(c * N + r0, 640)])

    @pl.when(s == 15)
    def _():
        pltpu.sync_copy(deg_sh.at[pl.ds(9600, 400)], zb.at[pl.ds(0, 400)])
        pltpu.sync_copy(zb.at[pl.ds(0, 400)],
                        degp_hbm.at[pl.ds(c * N + 9600, 400)])


_sc_deg = pl.kernel(
    _sc_deg_body,
    out_type=jax.ShapeDtypeStruct((2 * N,), jnp.float32),
    mesh=_mesh,
    scratch_types=[pltpu.VMEM((KE,), jnp.float32),
                   pltpu.VMEM((NCH, KE), jnp.int32),
                   pltpu.VMEM((640,), jnp.float32),
                   pltpu.VMEM_SHARED((N + 16,), jnp.float32),
                   pltpu.SemaphoreType.DMA],
    compiler_params=pltpu.CompilerParams(use_tc_tiling_on_sc=False),
)


def _zero_g0(g0):
    def zrow(r, carry):
        for cc in range(QW // 16):
            g0[r, pl.ds(cc * 16, 16)] = jnp.zeros((16,), jnp.float32)
        return carry

    lax.fori_loop(0, KE, zrow, 0)


def _chunked(base, total, fn):
    off = 0
    while off < total:
        n = min(KE, total - off)
        fn(base + off, n)
        off += n


def _acc_rows(s, fn):
    r0 = s * 640

    @pl.when(s < 15)
    def _():
        _chunked(r0, 640, fn)

    @pl.when(s == 15)
    def _():
        _chunked(9600, 400, fn)


def _agg_pass(q_hbm, out_hbm, sb2, db2, gbufs, gsems, ssems, acc_sh, s):
    g0 = gbufs[0]
    _zero_g0(g0)
    _acc_rows(s, lambda r, n: pltpu.sync_copy(
        g0.at[pl.ds(0, n)], acc_sh.at[pl.ds(r, n)]))
    plsc.subcore_barrier()

    for b in range(NBUF):
        pltpu.async_copy(q_hbm.at[sb2.at[b]], gbufs[b], gsems[b])

    def group(j, carry):
        descs = []
        for b in range(NBUF):
            k = NBUF * j + b
            pltpu.make_async_copy(q_hbm.at[sb2.at[0]], gbufs[b],
                                  gsems[b]).wait()
            descs.append(pltpu.async_copy(
                gbufs[b], acc_sh.at[db2.at[k]], ssems[b], add=True))
        for b in range(NBUF):
            k = NBUF * j + b
            descs[b].wait()

            @pl.when(k + NBUF < NCH)
            def _(b=b, k=k):
                pltpu.async_copy(q_hbm.at[sb2.at[k + NBUF]], gbufs[b],
                                 gsems[b])
        return carry

    lax.fori_loop(0, NGRP, group, 0)
    edescs = []
    for b in range(NEPI):
        pltpu.make_async_copy(q_hbm.at[sb2.at[0]], gbufs[b], gsems[b]).wait()
        edescs.append(pltpu.async_copy(
            gbufs[b], acc_sh.at[db2.at[NGRP * NBUF + b]], ssems[b], add=True))
    for d in edescs:
        d.wait()
    plsc.subcore_barrier()

    def wb(r, n):
        pltpu.sync_copy(acc_sh.at[pl.ds(r, n)], g0.at[pl.ds(0, n)])
        pltpu.sync_copy(g0.at[pl.ds(0, n)], out_hbm.at[pl.ds(r, n)])

    _acc_rows(s, wb)


def _edge_body(src3d_hbm, dst3d_hbm, q0_hbm, q1_hbm, q2_hbm, q3_hbm,
               a0_hbm, a1_hbm, a2_hbm, a3_hbm,
               sb2, db2, g0, g1, g2, g3, g4, g5, acc_sh,
               gs0, gs1, gs2, gs3, gs4, gs5,
               ss0, ss1, ss2, ss3, ss4, ss5):
    c = lax.axis_index("c")
    s = lax.axis_index("s")
    gbufs = (g0, g1, g2, g3, g4, g5)
    gsems = (gs0, gs1, gs2, gs3, gs4, gs5)
    ssems = (ss0, ss1, ss2, ss3, ss4, ss5)
    pltpu.sync_copy(src3d_hbm.at[s], sb2)
    pltpu.sync_copy(dst3d_hbm.at[s], db2)

    @pl.when(c == 0)
    def _():
        _agg_pass(q0_hbm, a0_hbm, sb2, db2, gbufs, gsems, ssems, acc_sh, s)
        plsc.subcore_barrier()
        _agg_pass(q1_hbm, a1_hbm, sb2, db2, gbufs, gsems, ssems, acc_sh, s)

    @pl.when(c == 1)
    def _():
        _agg_pass(q2_hbm, a2_hbm, sb2, db2, gbufs, gsems, ssems, acc_sh, s)
        plsc.subcore_barrier()
        _agg_pass(q3_hbm, a3_hbm, sb2, db2, gbufs, gsems, ssems, acc_sh, s)


_edge_scratch = [pltpu.VMEM((NCH, KE), jnp.int32),
                 pltpu.VMEM((NCH, KE), jnp.int32)] + \
                [pltpu.VMEM((KE, QW), jnp.float32)] * NBUF + \
                [pltpu.VMEM_SHARED((N + 16, QW), jnp.float32)] + \
                [pltpu.SemaphoreType.DMA] * (2 * NBUF)

_edge_agg = pl.kernel(
    _edge_body,
    out_type=[jax.ShapeDtypeStruct((N, QW), jnp.float32)] * 4,
    mesh=_mesh,
    scratch_types=_edge_scratch,
    compiler_params=pltpu.CompilerParams(use_tc_tiling_on_sc=False),
)


BLK = 1000
NB = N // BLK


def _prescale_body(x_ref, emb_ref, dpT_ref, s_ref, qa_ref, qb_ref):
    vids = lax.broadcasted_iota(jnp.int32, (BLK, 128), 1)
    h = jnp.zeros((BLK, EMB), jnp.float32)
    for f in range(NF):
        oh = (x_ref[:, f:f + 1] == vids).astype(jnp.float32)
        h = h + jnp.dot(oh, emb_ref[f])
    d = dpT_ref[:, 0:1] + dpT_ref[:, 1:2]
    s = lax.rsqrt(1.0 + d)
    s_ref[...] = s
    q = h * s
    qa_ref[...] = q[:, :QW]
    qb_ref[...] = q[:, QW:]


def _tc_prescale(x, emb_pad, dpT):
    return pl.pallas_call(
        _prescale_body,
        grid=(NB,),
        in_specs=[pl.BlockSpec((BLK, NF), lambda i: (i, 0)),
                  pl.BlockSpec((NF, 128, EMB), lambda i: (0, 0, 0)),
                  pl.BlockSpec((BLK, 2), lambda i: (i, 0))],
        out_specs=[pl.BlockSpec((BLK, 1), lambda i: (i, 0)),
                   pl.BlockSpec((BLK, QW), lambda i: (i, 0)),
                   pl.BlockSpec((BLK, QW), lambda i: (i, 0))],
        out_shape=[jax.ShapeDtypeStruct((N, 1), jnp.float32),
                   jax.ShapeDtypeStruct((N, QW), jnp.float32),
                   jax.ShapeDtypeStruct((N, QW), jnp.float32)],
    )(x, emb_pad, dpT)


def _layer0_body(a0, a1, q0, q1, s_ref, w_ref, b_ref, o0, o1, o2, o3):
    s = s_ref[...]
    z = jnp.concatenate([a0[...] + q0[...], a1[...] + q1[...]], axis=1) * s
    h = jnp.maximum(jnp.dot(z, w_ref[...]) + b_ref[...], 0.0)
    qn = h * s
    o0[...] = qn[:, 0 * QW:1 * QW]
    o1[...] = qn[:, 1 * QW:2 * QW]
    o2[...] = qn[:, 2 * QW:3 * QW]
    o3[...] = qn[:, 3 * QW:4 * QW]


def _layer_body(a0, a1, a2, a3, q0, q1, q2, q3, s_ref, w_ref, b_ref,
                o0, o1, o2, o3):
    s = s_ref[...]
    z = jnp.concatenate([a0[...] + q0[...], a1[...] + q1[...],
                         a2[...] + q2[...], a3[...] + q3[...]], axis=1) * s
    h = jnp.maximum(jnp.dot(z, w_ref[...]) + b_ref[...], 0.0)
    qn = h * s
    o0[...] = qn[:, 0 * QW:1 * QW]
    o1[...] = qn[:, 1 * QW:2 * QW]
    o2[...] = qn[:, 2 * QW:3 * QW]
    o3[...] = qn[:, 3 * QW:4 * QW]


_qspec = pl.BlockSpec((BLK, QW), lambda i: (i, 0))


def _tc_layer0(aq, qq, s, wm, bias):
    return pl.pallas_call(
        _layer0_body,
        grid=(NB,),
        in_specs=[_qspec] * 4 + [
            pl.BlockSpec((BLK, 1), lambda i: (i, 0)),
            pl.BlockSpec((EMB, HID), lambda i: (0, 0)),
            pl.BlockSpec((1, HID), lambda i: (0, 0))],
        out_specs=[_qspec] * 4,
        out_shape=[jax.ShapeDtypeStruct((N, QW), jnp.float32)] * 4,
    )(*aq, *qq, s, wm, bias.reshape(1, HID))


def _tc_layer(aq, qq, s, wm, bias):
    return pl.pallas_call(
        _layer_body,
        grid=(NB,),
        in_specs=[_qspec] * 8 + [
            pl.BlockSpec((BLK, 1), lambda i: (i, 0)),
            pl.BlockSpec((HID, HID), lambda i: (0, 0)),
            pl.BlockSpec((1, HID), lambda i: (0, 0))],
        out_specs=[_qspec] * 4,
        out_shape=[jax.ShapeDtypeStruct((N, QW), jnp.float32)] * 4,
    )(*aq, *qq, s, wm, bias.reshape(1, HID))


def _readout_body(q0, q1, q2, q3, s_ref, bidx_ref, linw_ref, linb_ref,
                  out_ref, sums_scr, cnts_scr):
    i = pl.program_id(0)
    s = s_ref[...]
    h3 = jnp.concatenate([q0[...], q1[...], q2[...], q3[...]], axis=1) / s
    gids = lax.broadcasted_iota(jnp.int32, (NG, BLK), 0)
    ohT = (bidx_ref[0] == gids).astype(jnp.float32)
    bs = jnp.dot(ohT, h3)
    bc = jnp.sum(ohT, axis=1, keepdims=True)

    @pl.when(i == 0)
    def _():
        sums_scr[...] = bs
        cnts_scr[...] = bc

    @pl.when(i > 0)
    def _():
        sums_scr[...] += bs
        cnts_scr[...] += bc

    @pl.when(i == NB - 1)
    def _():
        mean = sums_scr[...] / jnp.maximum(cnts_scr[...], 1.0)
        out_ref[...] = jax.nn.sigmoid(jnp.dot(mean, linw_ref[...])
                                      + linb_ref[...])


def _tc_readout(qq, s, bidx_3d, lin_W, lin_b):
    return pl.pallas_call(
        _readout_body,
        grid=(NB,),
        in_specs=[_qspec] * 4 + [
            pl.BlockSpec((BLK, 1), lambda i: (i, 0)),
            pl.BlockSpec((1, 1, BLK), lambda i: (i, 0, 0)),
            pl.BlockSpec((HID, 1), lambda i: (0, 0)),
            pl.BlockSpec((1, 1), lambda i: (0, 0))],
        out_specs=pl.BlockSpec((NG, 1), lambda i: (0, 0)),
        out_shape=jax.ShapeDtypeStruct((NG, 1), jnp.float32),
        scratch_shapes=[pltpu.VMEM((NG, HID), jnp.float32),
                        pltpu.VMEM((NG, 1), jnp.float32)],
    )(*qq, s, bidx_3d, lin_W, lin_b.reshape(1, 1))


def kernel(x, edge_index, batch_idx, atom_emb, W0, b0, W1, b1, W2, b2,
           lin_W, lin_b):
    x = x.astype(jnp.int32)
    src = edge_index[0].astype(jnp.int32)
    dst = edge_index[1].astype(jnp.int32)
    emb_pad = jnp.pad(atom_emb, ((0, 0), (0, 128 - VOCAB), (0, 0)))

    srcr = src.reshape(NS, EPT)
    dstr = dst.reshape(NS, EPT)
    src3d = jnp.concatenate(
        [srcr, jnp.zeros((NS, EPAD), jnp.int32)], axis=1).reshape(NS, NCH, KE)
    dst3d = jnp.concatenate(
        [dstr, jnp.full((NS, EPAD), DUMP, jnp.int32)],
        axis=1).reshape(NS, NCH, KE)

    degp = _sc_deg(dst3d)
    s, q0a, q0b = _tc_prescale(x, emb_pad, jnp.transpose(degp.reshape(2, N)))

    zq = jnp.zeros((N, QW), jnp.float32)
    aq = _edge_agg(src3d, dst3d, q0a, q0b, zq, zq)
    carry = _tc_layer0(aq[:2], (q0a, q0b), s, W0, b0)
    for wm, bias in ((W1, b1), (W2, b2)):
        aq = _edge_agg(src3d, dst3d, *carry)
        carry = _tc_layer(aq, carry, s, wm, bias)
    q3 = carry

    out = _tc_readout(q3, s,
                      batch_idx.astype(jnp.int32).reshape(NB, 1, BLK),
                      lin_W, lin_b)
    return out
